# speedup vs baseline: 1.0671x; 1.0671x over previous
import functools

import jax
import jax.numpy as jnp
from jax.experimental import pallas as pl
from jax.experimental.pallas import tpu as pltpu

N = 10000
E = 160000
H = 4
HC = 256
G = 64
NEG_SLOPE = 0.2
EPS = 1e-5


def _mm_kernel(x_ref, w_ref, o_ref):
    o_ref[...] = jnp.dot(x_ref[...], w_ref[...], preferred_element_type=jnp.float32)


def _matmul(x, w, block_rows=512):
    m, k = x.shape
    k2, n = w.shape
    grid = (m // block_rows,)
    return pl.pallas_call(
        _mm_kernel,
        grid=grid,
        in_specs=[
            pl.BlockSpec((block_rows, k), lambda i: (i, 0)),
            pl.BlockSpec((k, n), lambda i: (0, 0)),
        ],
        out_specs=pl.BlockSpec((block_rows, n), lambda i: (i, 0)),
        out_shape=jax.ShapeDtypeStruct((m, n), jnp.float32),
    )(x, w)


def _gat_conv(x, src, dst, W, a_s, a_d, b):
    h = (x @ W).reshape(N, H, -1)
    al_s = jnp.sum(h * a_s[None], axis=-1)
    al_d = jnp.sum(h * a_d[None], axis=-1)
    e = jax.nn.leaky_relu(al_s[src] + al_d[dst], NEG_SLOPE)
    ex = jnp.exp(e)
    den = jax.ops.segment_sum(ex, dst, num_segments=N)
    msg = h[src] * ex[:, :, None]
    acc = jax.ops.segment_sum(msg, dst, num_segments=N)
    out = acc / (den[:, :, None] + 1e-16)
    return out.reshape(N, -1) + b


def _bn(x, g, be):
    mu = jnp.mean(x, axis=0)
    var = jnp.var(x, axis=0)
    return g * (x - mu) / jnp.sqrt(var + EPS) + be


def kernel(x, edge_index, batch, W1, as1, ad1, b1, g1, be1, W2, as2, ad2, b2, g2, be2, W3, as3, ad3, b3, g3, be3, fw1, fb1, fw2, fb2):
    loop = jnp.arange(N, dtype=edge_index.dtype)
    src = jnp.concatenate([edge_index[0], loop])
    dst = jnp.concatenate([edge_index[1], loop])
    h = jax.nn.relu(_bn(_gat_conv(x, src, dst, W1, as1, ad1, b1), g1, be1))
    h = jax.nn.relu(_bn(_gat_conv(h, src, dst, W2, as2, ad2, b2), g2, be2))
    h = jax.nn.relu(_bn(_gat_conv(h, src, dst, W3, as3, ad3, b3), g3, be3))
    sums = jax.ops.segment_sum(h, batch, num_segments=G)
    cnt = jax.ops.segment_sum(jnp.ones((N,), dtype=h.dtype), batch, num_segments=G)
    gm = sums / jnp.maximum(cnt, 1.0)[:, None]
    out = jax.nn.relu(_matmul(gm, fw1, block_rows=64) + fb1)
    out = _matmul(out, fw2, block_rows=64) + fb2
    return out


# speedup vs baseline: 9.1292x; 8.5548x over previous
import functools

import jax
import jax.numpy as jnp
from jax import lax
from jax.experimental import pallas as pl
from jax.experimental.pallas import tpu as pltpu
from jax.experimental.pallas import tpu_sc as plsc

N = 10000
NP = 10240
E = 160000
E2 = E + N
H = 4
G = 64
NEG_SLOPE = 0.2
EPS = 1e-5
F = 1024
NFB = 8
BM = 512
NGRID = NP // BM
NTILES = 16
EPT = 10752
NBATCH = 168
BB = 64
RPT = NP // NTILES


def _stage_a_body(x_ref, w_ref, amt_ref, h_ref, alt_ref):
    h = jnp.dot(x_ref[...], w_ref[...], preferred_element_type=jnp.float32)
    for fb in range(NFB):
        h_ref[fb] = h[:, fb * 128:(fb + 1) * 128]
    alt_ref[...] = lax.dot_general(
        amt_ref[...], h, (((1,), (1,)), ((), ())),
        precision=lax.Precision.HIGHEST,
        preferred_element_type=jnp.float32)


def _stage_a(x, w, amt):
    fin = x.shape[1]
    return pl.pallas_call(
        _stage_a_body,
        grid=(NGRID,),
        in_specs=[
            pl.BlockSpec((BM, fin), lambda i: (i, 0)),
            pl.BlockSpec((fin, F), lambda i: (0, 0)),
            pl.BlockSpec((NFB, F), lambda i: (0, 0)),
        ],
        out_specs=[
            pl.BlockSpec((NFB, BM, 128), lambda i: (0, i, 0)),
            pl.BlockSpec((NFB, BM), lambda i: (0, i)),
        ],
        out_shape=[
            jax.ShapeDtypeStruct((NFB, NP, 128), jnp.float32),
            jax.ShapeDtypeStruct((NFB, NP), jnp.float32),
        ],
    )(x, w, amt)


def _stats_body(hagg_ref, denp_ref, denr_ref, stats_ref):
    i = pl.program_id(0)
    den = jnp.sum(denp_ref[...], axis=1)
    denr_ref[...] = jnp.concatenate([den, jnp.zeros_like(den)], axis=0)

    @pl.when(i == 0)
    def _():
        stats_ref[...] = jnp.zeros_like(stats_ref)

    s = []
    q = []
    for fb in range(NFB):
        inv = 1.0 / (den[fb // 2] + 1e-16)
        z = hagg_ref[fb] * inv[:, None]
        s.append(jnp.sum(z, axis=0))
        q.append(jnp.sum(z * z, axis=0))
    stats_ref[...] += jnp.stack([jnp.stack(s), jnp.stack(q)])


def _stage_c1(hagg, denp):
    return pl.pallas_call(
        _stats_body,
        grid=(NGRID,),
        in_specs=[
            pl.BlockSpec((NFB, BM, 128), lambda i: (0, i, 0)),
            pl.BlockSpec((H, NTILES, BM), lambda i: (0, 0, i)),
        ],
        out_specs=[
            pl.BlockSpec((2 * H, BM), lambda i: (0, i)),
            pl.BlockSpec((2, NFB, 128), lambda i: (0, 0, 0)),
        ],
        out_shape=[
            jax.ShapeDtypeStruct((2 * H, NP), jnp.float32),
            jax.ShapeDtypeStruct((2, NFB, 128), jnp.float32),
        ],
    )(hagg, denp)


def _c2_body(hagg_ref, denr_ref, stats_ref, g_ref, be_ref, w_ref, amt_ref,
             hn_ref, altn_ref):
    mu = stats_ref[0] / N
    var = stats_ref[1] / N - mu * mu
    sc = g_ref[...] * lax.rsqrt(var + EPS)
    den = denr_ref[...]
    hn = jnp.zeros((BM, F), jnp.float32)
    altn = jnp.zeros((NFB, BM), jnp.float32)
    for fb in range(NFB):
        inv = 1.0 / (den[fb // 2] + 1e-16)
        z = hagg_ref[fb] * inv[:, None]
        xn = (z - mu[fb]) * sc[fb] + be_ref[fb]
        hr = jnp.maximum(xn, 0.0)
        hn = hn + jnp.dot(hr, w_ref[fb], preferred_element_type=jnp.float32)
        altn = altn + lax.dot_general(
            amt_ref[:, fb, :], hr, (((1,), (1,)), ((), ())),
            precision=lax.Precision.HIGHEST,
            preferred_element_type=jnp.float32)
    for fb in range(NFB):
        hn_ref[fb] = hn[:, fb * 128:(fb + 1) * 128]
    altn_ref[...] = altn


def _stage_c2(hagg, denr, stats, g, be, w, amt):
    return pl.pallas_call(
        _c2_body,
        grid=(NGRID,),
        in_specs=[
            pl.BlockSpec((NFB, BM, 128), lambda i: (0, i, 0)),
            pl.BlockSpec((2 * H, BM), lambda i: (0, i)),
            pl.BlockSpec((2, NFB, 128), lambda i: (0, 0, 0)),
            pl.BlockSpec((NFB, 128), lambda i: (0, 0)),
            pl.BlockSpec((NFB, 128), lambda i: (0, 0)),
            pl.BlockSpec((NFB, 128, F), lambda i: (0, 0, 0)),
            pl.BlockSpec((NFB, NFB, 128), lambda i: (0, 0, 0)),
        ],
        out_specs=[
            pl.BlockSpec((NFB, BM, 128), lambda i: (0, i, 0)),
            pl.BlockSpec((NFB, BM), lambda i: (0, i)),
        ],
        out_shape=[
            jax.ShapeDtypeStruct((NFB, NP, 128), jnp.float32),
            jax.ShapeDtypeStruct((NFB, NP), jnp.float32),
        ],
    )(hagg, denr, stats, g, be, w, amt)


def _c3_body(hagg_ref, denr_ref, stats_ref, g_ref, be_ref, b_ref,
             sums_ref, cnt_ref):
    i = pl.program_id(0)
    mu = stats_ref[0] / N
    var = stats_ref[1] / N - mu * mu
    sc = g_ref[...] * lax.rsqrt(var + EPS)
    den = denr_ref[...]
    gids = lax.broadcasted_iota(jnp.int32, (G, BM), 0)
    onehot = (b_ref[0] == gids).astype(jnp.float32)

    @pl.when(i == 0)
    def _():
        sums_ref[...] = jnp.zeros_like(sums_ref)
        cnt_ref[...] = jnp.zeros_like(cnt_ref)

    acc = []
    for fb in range(NFB):
        inv = 1.0 / (den[fb // 2] + 1e-16)
        z = hagg_ref[fb] * inv[:, None]
        xn = (z - mu[fb]) * sc[fb] + be_ref[fb]
        hr = jnp.maximum(xn, 0.0)
        acc.append(jnp.dot(onehot, hr, preferred_element_type=jnp.float32))
    sums_ref[...] += jnp.stack(acc)
    cnt_ref[...] += jnp.broadcast_to(
        jnp.sum(onehot, axis=1, keepdims=True), (G, 128))


def _stage_c3(hagg, denr, stats, g, be, batch2d):
    return pl.pallas_call(
        _c3_body,
        grid=(NGRID,),
        in_specs=[
            pl.BlockSpec((NFB, BM, 128), lambda i: (0, i, 0)),
            pl.BlockSpec((2 * H, BM), lambda i: (0, i)),
            pl.BlockSpec((2, NFB, 128), lambda i: (0, 0, 0)),
            pl.BlockSpec((NFB, 128), lambda i: (0, 0)),
            pl.BlockSpec((NFB, 128), lambda i: (0, 0)),
            pl.BlockSpec((1, 1, BM), lambda i: (i, 0, 0)),
        ],
        out_specs=[
            pl.BlockSpec((NFB, G, 128), lambda i: (0, 0, 0)),
            pl.BlockSpec((G, 128), lambda i: (0, 0)),
        ],
        out_shape=[
            jax.ShapeDtypeStruct((NFB, G, 128), jnp.float32),
            jax.ShapeDtypeStruct((G, 128), jnp.float32),
        ],
    )(hagg, denr, stats, g, be, batch2d)


def _d_body(sums_ref, cnt_ref, fw1_ref, fb1_ref, fw2_ref, fb2_ref, o_ref):
    c = 1.0 / jnp.maximum(cnt_ref[...], 1.0)
    o1 = jnp.zeros((G, 512), jnp.float32)
    for fb in range(NFB):
        gm = sums_ref[fb] * c
        o1 = o1 + jnp.dot(gm, fw1_ref[fb], preferred_element_type=jnp.float32)
    o1 = jnp.maximum(o1 + fb1_ref[...], 0.0)
    o_ref[...] = jnp.dot(o1, fw2_ref[...],
                         preferred_element_type=jnp.float32) + fb2_ref[...]


def _stage_d(sums, cnt, fw1r, fb1, fw2, fb2):
    return pl.pallas_call(
        _d_body,
        out_shape=jax.ShapeDtypeStruct((G, 256), jnp.float32),
    )(sums, cnt, fw1r, fb1.reshape(1, 512), fw2, fb2.reshape(1, 256))


def _sc_agg_wrapped(h_hbm, alt_hbm, src_hbm, dst_hbm, hagg_hbm, denp_hbm,
                    src_b, dst_b, als_v, ald_v, den_v, w_v, rows_v,
                    acc_sh, sem):
    c = lax.axis_index("c")
    s = lax.axis_index("s")

    for k in range(4):
        fb = 4 * c + k
        head = 2 * c + (k // 2)
        do_den = (k % 2 == 0)
        pltpu.sync_copy(alt_hbm.at[head], als_v)
        pltpu.sync_copy(alt_hbm.at[4 + head], ald_v)

        def zz(r, _):
            for gg in range(8):
                rows_v[r, pl.ds(gg * 16, 16)] = jnp.zeros((16,), jnp.float32)
            return 0
        lax.fori_loop(0, BB, zz, 0)
        for j in range(RPT // BB):
            pltpu.sync_copy(rows_v, acc_sh.at[pl.ds(s * RPT + j * BB, BB)])
        if do_den:
            def zd(r, _):
                den_v[pl.ds(r * 16, 16)] = jnp.zeros((16,), jnp.float32)
                return 0
            lax.fori_loop(0, NP // 16, zd, 0)
        plsc.subcore_barrier()

        def batch_body(b, _):
            pltpu.sync_copy(src_hbm.at[s].at[b], src_b)
            pltpu.sync_copy(dst_hbm.at[s].at[b], dst_b)
            pltpu.async_copy(h_hbm.at[fb].at[src_b], rows_v, sem).wait()
            for g in range(BB // 16):
                s16 = src_b[pl.ds(g * 16, 16)]
                d16 = dst_b[pl.ds(g * 16, 16)]
                a = plsc.load_gather(als_v, [s16])
                bb = plsc.load_gather(ald_v, [d16])
                t = a + bb
                t = jnp.where(t >= 0.0, t, t * NEG_SLOPE)
                w = jnp.exp(t)
                gid = s * EPT + b * BB + g * 16 + lax.iota(jnp.int32, 16)
                w = jnp.where(gid < E2, w, 0.0)
                w_v[pl.ds(g * 16, 16)] = w
                if do_den:
                    plsc.addupdate_scatter(den_v, [d16], w)

            def edge_body(e, _):
                wspl = plsc.load_gather(w_v, [jnp.full((16,), e, jnp.int32)])
                for j in range(8):
                    rows_v[e, pl.ds(j * 16, 16)] = (
                        rows_v[e, pl.ds(j * 16, 16)] * wspl)
                return 0
            lax.fori_loop(0, BB, edge_body, 0)
            pltpu.sync_copy(rows_v, acc_sh.at[dst_b], add=True)
            return 0
        lax.fori_loop(0, NBATCH, batch_body, 0)
        plsc.subcore_barrier()
        pltpu.sync_copy(acc_sh.at[pl.ds(s * RPT, RPT)],
                        hagg_hbm.at[fb].at[pl.ds(s * RPT, RPT)])
        if do_den:
            pltpu.sync_copy(den_v, denp_hbm.at[head].at[s])
        plsc.subcore_barrier()


def _sc_agg(h, alt, srcp, dstp):
    f = pl.kernel(
        _sc_agg_wrapped,
        mesh=plsc.VectorSubcoreMesh(core_axis_name="c", subcore_axis_name="s"),
        compiler_params=pltpu.CompilerParams(needs_layout_passes=False),
        out_type=[
            jax.ShapeDtypeStruct((NFB, NP, 128), jnp.float32),
            jax.ShapeDtypeStruct((H, NTILES, NP), jnp.float32),
        ],
        scratch_types=[
            pltpu.VMEM((BB,), jnp.int32),
            pltpu.VMEM((BB,), jnp.int32),
            pltpu.VMEM((NP,), jnp.float32),
            pltpu.VMEM((NP,), jnp.float32),
            pltpu.VMEM((NP,), jnp.float32),
            pltpu.VMEM((BB,), jnp.float32),
            pltpu.VMEM((BB, 128), jnp.float32),
            pltpu.VMEM_SHARED((NP, 128), jnp.float32),
            pltpu.SemaphoreType.DMA,
        ],
    )
    return f(h, alt, srcp, dstp)


def _amat_t(a_s, a_d):
    din = a_s.shape[1] * H
    headof = jnp.arange(din, dtype=jnp.int32) // a_s.shape[1]
    rows_s = (headof[None, :] == jnp.arange(H, dtype=jnp.int32)[:, None])
    ms = rows_s * a_s.reshape(-1)[None, :]
    md = rows_s * a_d.reshape(-1)[None, :]
    return jnp.concatenate([ms, md], axis=0).astype(jnp.float32)


def kernel(x, edge_index, batch, W1, as1, ad1, b1, g1, be1, W2, as2, ad2, b2,
           g2, be2, W3, as3, ad3, b3, g3, be3, fw1, fb1, fw2, fb2):
    loop = jnp.arange(N, dtype=edge_index.dtype)
    src = jnp.concatenate([edge_index[0], loop])
    dst = jnp.concatenate([edge_index[1], loop])
    pad = NTILES * EPT - E2
    srcp = jnp.concatenate([src, jnp.zeros((pad,), src.dtype)])
    dstp = jnp.concatenate([dst, jnp.zeros((pad,), dst.dtype)])
    srcp = srcp.reshape(NTILES, NBATCH, BB)
    dstp = dstp.reshape(NTILES, NBATCH, BB)
    xp = jnp.concatenate(
        [x, jnp.zeros((NP - N, x.shape[1]), jnp.float32)], axis=0)
    batchp = jnp.concatenate(
        [batch.astype(jnp.int32), jnp.full((NP - N,), -1, jnp.int32)])
    batch2d = batchp.reshape(NGRID, 1, BM)

    amt1 = _amat_t(as1, ad1)
    cmat2 = jnp.dot(_amat_t(as2, ad2), W2.T, precision=lax.Precision.HIGHEST)
    cmat3 = jnp.dot(_amat_t(as3, ad3), W3.T, precision=lax.Precision.HIGHEST)
    w2r = W2.reshape(NFB, 128, F)
    w3r = W3.reshape(NFB, 128, F)
    fw1r = fw1.reshape(NFB, 128, 512)

    h, alt = _stage_a(xp, W1, amt1)
    params = [(g1, be1, w2r, cmat2.reshape(NFB, NFB, 128)),
              (g2, be2, w3r, cmat3.reshape(NFB, NFB, 128))]
    for g, be, wr, amtr in params:
        hagg, denp = _sc_agg(h, alt, srcp, dstp)
        denr, stats = _stage_c1(hagg, denp)
        h, alt = _stage_c2(hagg, denr, stats, g.reshape(NFB, 128),
                           be.reshape(NFB, 128), wr, amtr)
    hagg, denp = _sc_agg(h, alt, srcp, dstp)
    denr, stats = _stage_c1(hagg, denp)
    sums, cnt = _stage_c3(hagg, denr, stats, g3.reshape(NFB, 128),
                          be3.reshape(NFB, 128), batch2d)
    return _stage_d(sums, cnt, fw1r, fb1, fw2, fb2)
